# baseline (speedup 1.0000x reference)
import functools
import jax
import jax.numpy as jnp
from jax import lax
from jax.experimental import pallas as pl
from jax.experimental.pallas import tpu as pltpu
from jax.experimental.pallas import tpu_sc as plsc

_NUM_EXPERTS = 16
_COEF = 0.01
_EPS = 1e-10
_NW = 32


def _matmul_body(x_ref, w_ref, out_ref):
    out_ref[...] = jnp.dot(x_ref[...], w_ref[...],
                           preferred_element_type=jnp.float32)


def _tc_logits(x, w_gate):
    n_tokens, d_model = x.shape
    tile = 1024
    return pl.pallas_call(
        _matmul_body,
        grid=(n_tokens // tile,),
        in_specs=[
            pl.BlockSpec((tile, d_model), lambda i: (i, 0)),
            pl.BlockSpec((d_model, _NUM_EXPERTS), lambda i: (0, 0)),
        ],
        out_specs=pl.BlockSpec((tile, _NUM_EXPERTS), lambda i: (i, 0)),
        out_shape=jax.ShapeDtypeStruct((n_tokens, _NUM_EXPERTS), jnp.float32),
    )(x, w_gate)


def _make_sc_route(n_tokens):
    tpw = n_tokens // _NW
    mesh = plsc.VectorSubcoreMesh(core_axis_name="c", subcore_axis_name="s")

    @functools.partial(
        pl.kernel, mesh=mesh,
        out_type=[
            jax.ShapeDtypeStruct((n_tokens, _NUM_EXPERTS), jnp.float32),
            jax.ShapeDtypeStruct((2 * _NW, _NUM_EXPERTS), jnp.float32),
        ],
        scratch_types=[
            pltpu.VMEM((tpw, _NUM_EXPERTS), jnp.float32),
            pltpu.VMEM((tpw, _NUM_EXPERTS), jnp.float32),
            pltpu.VMEM((1, _NUM_EXPERTS), jnp.float32),
        ],
        compiler_params=pltpu.CompilerParams(needs_layout_passes=False),
    )
    def route(logits_hbm, gates_hbm, parts_hbm, log_v, out_v, part_v):
        wid = lax.axis_index("s") * 2 + lax.axis_index("c")
        base = wid * tpw
        pltpu.sync_copy(logits_hbm.at[pl.ds(base, tpw), :], log_v)

        zeros = jnp.zeros((16,), jnp.float32)
        ones = jnp.ones((16,), jnp.float32)
        neg_inf = jnp.full((16,), -jnp.inf, jnp.float32)

        def body(t, carry):
            imp, ld = carry
            lv = log_v[t, :]
            m1 = jnp.broadcast_to(jnp.max(lv), (16,))
            eq1 = lv == m1
            c1 = plsc.cumsum(eq1.astype(jnp.int32))
            first1 = eq1 & (c1 == 1)
            masked = jnp.where(first1, neg_inf, lv)
            m2 = jnp.broadcast_to(jnp.max(masked), (16,))
            eq2 = masked == m2
            c2 = plsc.cumsum(eq2.astype(jnp.int32))
            first2 = eq2 & (c2 == 1)
            d = jnp.exp(m2 - m1)
            s = ones + d
            g1 = ones / s
            g2 = d / s
            gates = jnp.where(first1, g1, jnp.where(first2, g2, zeros))
            out_v[t, :] = gates
            return (imp + gates,
                    ld + jnp.where(gates > 0.0, ones, zeros))

        imp, ld = lax.fori_loop(0, tpw, body, (zeros, zeros))
        pltpu.sync_copy(out_v, gates_hbm.at[pl.ds(base, tpw), :])
        part_v[0, :] = imp
        pltpu.sync_copy(part_v, parts_hbm.at[pl.ds(wid, 1), :])
        part_v[0, :] = ld
        pltpu.sync_copy(part_v, parts_hbm.at[pl.ds(_NW + wid, 1), :])

    return route


def _combine_body(p_ref, aux_ref):
    imp = jnp.sum(p_ref[0:_NW, :], axis=0)
    ld = jnp.sum(p_ref[_NW:2 * _NW, :], axis=0)
    ne = float(_NUM_EXPERTS)
    imp_mean = jnp.sum(imp) / ne
    ld_mean = jnp.sum(ld) / ne
    imp_var = jnp.sum((imp - imp_mean) ** 2) / (ne - 1.0)
    ld_var = jnp.sum((ld - ld_mean) ** 2) / (ne - 1.0)
    aux_ref[0, 0] = _COEF * (imp_var / (imp_mean * imp_mean + _EPS)
                             + ld_var / (ld_mean * ld_mean + _EPS))


def _tc_combine(parts):
    return pl.pallas_call(
        _combine_body,
        out_specs=pl.BlockSpec(memory_space=pltpu.SMEM),
        out_shape=jax.ShapeDtypeStruct((1, 1), jnp.float32),
    )(parts)


def kernel(x, w_gate):
    n_tokens, _ = x.shape
    logits = _tc_logits(x, w_gate)
    gates, parts = _make_sc_route(n_tokens)(logits)
    aux = _tc_combine(parts)
    return gates, aux.reshape(())


# speedup vs baseline: 1.0062x; 1.0062x over previous
import functools
import jax
import jax.numpy as jnp
from jax import lax
from jax.experimental import pallas as pl
from jax.experimental.pallas import tpu as pltpu
from jax.experimental.pallas import tpu_sc as plsc

_NUM_EXPERTS = 16
_COEF = 0.01
_EPS = 1e-10
_NW = 32
_L = 16


def _matmul_body(x_ref, w_ref, out_ref):
    out_ref[...] = jnp.dot(x_ref[...], w_ref[...],
                           preferred_element_type=jnp.float32)


def _tc_logits(x, w_gate):
    n_tokens, d_model = x.shape
    tile = 2048
    return pl.pallas_call(
        _matmul_body,
        grid=(n_tokens // tile,),
        in_specs=[
            pl.BlockSpec((tile, d_model), lambda i: (i, 0)),
            pl.BlockSpec((d_model, _NUM_EXPERTS), lambda i: (0, 0)),
        ],
        out_specs=pl.BlockSpec((tile, _NUM_EXPERTS), lambda i: (i, 0)),
        out_shape=jax.ShapeDtypeStruct((n_tokens, _NUM_EXPERTS), jnp.float32),
    )(x, w_gate)


def _make_sc_route(n_tokens):
    tpw = n_tokens // _NW
    n_groups = tpw // _L
    mesh = plsc.VectorSubcoreMesh(core_axis_name="c", subcore_axis_name="s")

    @functools.partial(
        pl.kernel, mesh=mesh,
        out_type=[
            jax.ShapeDtypeStruct((n_tokens, _NUM_EXPERTS), jnp.float32),
            jax.ShapeDtypeStruct((2 * _NW, _NUM_EXPERTS), jnp.float32),
        ],
        scratch_types=[
            pltpu.VMEM((tpw, _NUM_EXPERTS), jnp.float32),
            pltpu.VMEM((tpw, _NUM_EXPERTS), jnp.float32),
            pltpu.VMEM((2, _NUM_EXPERTS), jnp.float32),
        ],
        compiler_params=pltpu.CompilerParams(needs_layout_passes=False),
    )
    def route(logits_hbm, gates_hbm, parts_hbm, log_v, out_v, part_v):
        wid = lax.axis_index("s") * 2 + lax.axis_index("c")
        base = wid * tpw
        pltpu.sync_copy(logits_hbm.at[pl.ds(base, tpw), :], log_v)

        lane = lax.iota(jnp.int32, _L)
        zeros = jnp.zeros((_L,), jnp.float32)
        ones = jnp.ones((_L,), jnp.float32)
        neg_inf = jnp.full((_L,), -jnp.inf, jnp.float32)

        part_v[0, :] = zeros
        part_v[1, :] = zeros

        def group_body(g, _):
            row = g * _L + lane
            m1 = plsc.load_gather(log_v, [row, jnp.zeros((_L,), jnp.int32)])
            e1 = jnp.zeros((_L,), jnp.int32)
            m2 = neg_inf
            e2 = jnp.zeros((_L,), jnp.int32)
            for e in range(1, _NUM_EXPERTS):
                ev = jnp.full((_L,), e, jnp.int32)
                lv = plsc.load_gather(log_v, [row, ev])
                new1 = lv > m1
                new2 = jnp.logical_and(jnp.logical_not(new1), lv > m2)
                m2 = jnp.where(new1, m1, jnp.where(new2, lv, m2))
                e2 = jnp.where(new1, e1, jnp.where(new2, ev, e2))
                m1 = jnp.where(new1, lv, m1)
                e1 = jnp.where(new1, ev, e1)

            d = jnp.exp(m2 - m1)
            s = ones + d
            g1 = ones / s
            g2 = d / s

            for j in range(_L):
                out_v[g * _L + j, :] = zeros
            plsc.addupdate_scatter(out_v, [row, e1], g1)
            plsc.addupdate_scatter(out_v, [row, e2], g2)

            plsc.addupdate_scatter(part_v, [jnp.zeros((_L,), jnp.int32), e1],
                                   g1)
            plsc.addupdate_scatter(part_v, [jnp.zeros((_L,), jnp.int32), e2],
                                   g2)
            plsc.addupdate_scatter(part_v, [jnp.ones((_L,), jnp.int32), e1],
                                   ones)
            plsc.addupdate_scatter(part_v, [jnp.ones((_L,), jnp.int32), e2],
                                   jnp.where(g2 > 0.0, ones, zeros))
            return 0

        lax.fori_loop(0, n_groups, group_body, 0)
        pltpu.sync_copy(out_v, gates_hbm.at[pl.ds(base, tpw), :])
        pltpu.sync_copy(part_v.at[pl.ds(0, 1), :],
                        parts_hbm.at[pl.ds(wid, 1), :])
        pltpu.sync_copy(part_v.at[pl.ds(1, 1), :],
                        parts_hbm.at[pl.ds(_NW + wid, 1), :])

    return route


def _combine_body(p_ref, aux_ref):
    imp = jnp.sum(p_ref[0:_NW, :], axis=0)
    ld = jnp.sum(p_ref[_NW:2 * _NW, :], axis=0)
    ne = float(_NUM_EXPERTS)
    imp_mean = jnp.sum(imp) / ne
    ld_mean = jnp.sum(ld) / ne
    imp_var = jnp.sum((imp - imp_mean) ** 2) / (ne - 1.0)
    ld_var = jnp.sum((ld - ld_mean) ** 2) / (ne - 1.0)
    aux_ref[0, 0] = _COEF * (imp_var / (imp_mean * imp_mean + _EPS)
                             + ld_var / (ld_mean * ld_mean + _EPS))


def _tc_combine(parts):
    return pl.pallas_call(
        _combine_body,
        out_specs=pl.BlockSpec(memory_space=pltpu.SMEM),
        out_shape=jax.ShapeDtypeStruct((1, 1), jnp.float32),
    )(parts)


def kernel(x, w_gate):
    n_tokens, _ = x.shape
    logits = _tc_logits(x, w_gate)
    gates, parts = _make_sc_route(n_tokens)(logits)
    aux = _tc_combine(parts)
    return gates, aux.reshape(())
